# baseline (speedup 1.0000x reference)
import functools

import jax
import jax.numpy as jnp
from jax import lax
from jax.experimental import pallas as pl
from jax.experimental.pallas import tpu as pltpu
from jax.experimental.pallas import tpu_sc as plsc

_NUM_CLASSES = 100000
_D = 64
_B = 16384
_LAMBDA_C = 1.0

_L = 16
_NC = 2
_NS = 16
_NW = _NC * _NS
_DPW = _D // _NW
_BQ = _B // 4


def _make_sc_kernel():
    mesh = plsc.VectorSubcoreMesh(core_axis_name="c", subcore_axis_name="s")

    @functools.partial(
        pl.kernel,
        mesh=mesh,
        compiler_params=pltpu.CompilerParams(needs_layout_passes=False),
        out_type=jax.ShapeDtypeStruct((_NW, _L), jnp.float32),
        scratch_types=[
            pltpu.VMEM((_NUM_CLASSES,), jnp.float32),
            pltpu.VMEM((_BQ,), jnp.int32),
            pltpu.VMEM((_BQ,), jnp.int32),
            pltpu.VMEM((_BQ,), jnp.float32),
            pltpu.VMEM((_BQ,), jnp.float32),
            pltpu.VMEM((_L,), jnp.float32),
            pltpu.SemaphoreType.DMA,
            pltpu.SemaphoreType.DMA,
            pltpu.SemaphoreType.DMA,
        ],
    )
    def sc_kernel(ct_hbm, y_hbm, ht_hbm, out_hbm,
                  crow, yq0, yq1, hq0, hq1, acc_v, csem, qsem0, qsem1):
        yqs = (yq0, yq1)
        hqs = (hq0, hq1)
        qsems = (qsem0, qsem1)
        wid = lax.axis_index("s") * _NC + lax.axis_index("c")

        def issue_q(d, q):
            b = q % 2
            return [
                pltpu.async_copy(y_hbm.at[pl.ds(q * _BQ, _BQ)], yqs[b],
                                 qsems[b]),
                pltpu.async_copy(ht_hbm.at[d, pl.ds(q * _BQ, _BQ)], hqs[b],
                                 qsems[b]),
            ]

        zero = jnp.zeros((_L,), jnp.float32)
        accs = (zero,) * 8

        d0 = wid * _DPW
        ccopy = pltpu.async_copy(ct_hbm.at[d0], crow, csem)
        pending = issue_q(d0, 0)

        for k in range(_DPW):
            d = d0 + k
            ccopy.wait()
            for q in range(4):
                b = q % 2
                if q + 1 < 4:
                    nxt = issue_q(d, q + 1)
                elif k + 1 < _DPW:
                    nxt = issue_q(d + 1, 0)
                else:
                    nxt = None
                for c in pending:
                    c.wait()
                pending = nxt

                yq, hq = yqs[b], hqs[b]

                @plsc.parallel_loop(0, _BQ, step=8 * _L, unroll=2, carry=accs)
                def accs(o, accs, yq=yq, hq=hq):
                    a = list(accs)
                    for u in range(8):
                        iv = yq[pl.ds(o + u * _L, _L)]
                        gv = plsc.load_gather(crow, [iv])
                        hv = hq[pl.ds(o + u * _L, _L)]
                        dv = hv - gv
                        a[u] = a[u] + dv * dv
                    return tuple(a)

            if k + 1 < _DPW:
                ccopy = pltpu.async_copy(ct_hbm.at[d + 1], crow, csem)

        a0, a1, a2, a3, a4, a5, a6, a7 = accs
        acc_v[...] = ((a0 + a1) + (a2 + a3)) + ((a4 + a5) + (a6 + a7))
        pltpu.sync_copy(acc_v, out_hbm.at[wid])

    return sc_kernel


_sc_kernel = _make_sc_kernel()


def kernel(y, hidden, centers):
    ct = jnp.transpose(centers)
    ht = jnp.transpose(hidden)
    partials = _sc_kernel(ct, y.astype(jnp.int32), ht)
    return (_LAMBDA_C / 2.0 / _B) * jnp.sqrt(jnp.sum(partials))


# speedup vs baseline: 1.0202x; 1.0202x over previous
import functools

import jax
import jax.numpy as jnp
from jax import lax
from jax.experimental import pallas as pl
from jax.experimental.pallas import tpu as pltpu
from jax.experimental.pallas import tpu_sc as plsc

_NUM_CLASSES = 100000
_D = 64
_B = 16384
_LAMBDA_C = 1.0

_L = 16
_NC = 2
_NS = 16
_NW = _NC * _NS
_DPW = _D // _NW
_BQ = _B // 4


def _make_sc_kernel():
    mesh = plsc.VectorSubcoreMesh(core_axis_name="c", subcore_axis_name="s")

    @functools.partial(
        pl.kernel,
        mesh=mesh,
        compiler_params=pltpu.CompilerParams(needs_layout_passes=False),
        out_type=jax.ShapeDtypeStruct((_NW, _L), jnp.float32),
        scratch_types=[
            pltpu.VMEM((_NUM_CLASSES,), jnp.float32),
            pltpu.VMEM((_BQ,), jnp.int32),
            pltpu.VMEM((_BQ,), jnp.int32),
            pltpu.VMEM((_BQ,), jnp.float32),
            pltpu.VMEM((_BQ,), jnp.float32),
            pltpu.VMEM((_L,), jnp.float32),
            pltpu.SemaphoreType.DMA,
            pltpu.SemaphoreType.DMA,
            pltpu.SemaphoreType.DMA,
        ],
    )
    def sc_kernel(ct_hbm, y_hbm, ht_hbm, out_hbm,
                  crow, yq0, yq1, hq0, hq1, acc_v, csem, qsem0, qsem1):
        yqs = (yq0, yq1)
        hqs = (hq0, hq1)
        qsems = (qsem0, qsem1)
        wid = lax.axis_index("s") * _NC + lax.axis_index("c")

        def issue_q(d, q):
            b = q % 2
            return [
                pltpu.async_copy(y_hbm.at[pl.ds(q * _BQ, _BQ)], yqs[b],
                                 qsems[b]),
                pltpu.async_copy(ht_hbm.at[d, pl.ds(q * _BQ, _BQ)], hqs[b],
                                 qsems[b]),
            ]

        zero = jnp.zeros((_L,), jnp.float32)
        accs = (zero,) * 8

        d0 = wid * _DPW
        ccopy = pltpu.async_copy(ct_hbm.at[d0], crow, csem)
        pending = issue_q(d0, 0)

        for k in range(_DPW):
            d = d0 + k
            ccopy.wait()
            for q in range(4):
                b = q % 2
                if q + 1 < 4:
                    nxt = issue_q(d, q + 1)
                elif k + 1 < _DPW:
                    nxt = issue_q(d + 1, 0)
                else:
                    nxt = None
                for c in pending:
                    c.wait()
                pending = nxt

                yq, hq = yqs[b], hqs[b]

                @plsc.parallel_loop(0, _BQ, step=8 * _L, carry=accs)
                def accs(o, accs, yq=yq, hq=hq):
                    a = list(accs)
                    for u in range(8):
                        iv = yq[pl.ds(o + u * _L, _L)]
                        gv = plsc.load_gather(crow, [iv])
                        hv = hq[pl.ds(o + u * _L, _L)]
                        dv = hv - gv
                        a[u] = a[u] + dv * dv
                    return tuple(a)

            if k + 1 < _DPW:
                ccopy = pltpu.async_copy(ct_hbm.at[d + 1], crow, csem)

        a0, a1, a2, a3, a4, a5, a6, a7 = accs
        acc_v[...] = ((a0 + a1) + (a2 + a3)) + ((a4 + a5) + (a6 + a7))
        pltpu.sync_copy(acc_v, out_hbm.at[wid])

    return sc_kernel


_sc_kernel = _make_sc_kernel()


def kernel(y, hidden, centers):
    ct = jnp.transpose(centers)
    ht = jnp.transpose(hidden)
    partials = _sc_kernel(ct, y.astype(jnp.int32), ht)
    return (_LAMBDA_C / 2.0 / _B) * jnp.sqrt(jnp.sum(partials))
